# baseline (speedup 1.0000x reference)
import functools

import jax
import jax.numpy as jnp
from jax import lax
from jax.experimental import pallas as pl
from jax.experimental.pallas import tpu as pltpu
from jax.experimental.pallas import tpu_sc as plsc

_NUM_CORES = 2
_NUM_SUBCORES = 16
_NUM_WORKERS = _NUM_CORES * _NUM_SUBCORES


def _pe_broadcast_body(B, T, D, rows_per_worker, chunk_rows, pe_hbm, out_hbm,
                       buf0, buf1, rsem0, rsem1):
    wid = lax.axis_index("s") * _NUM_CORES + lax.axis_index("c")
    base = wid * rows_per_worker
    nchunks = rows_per_worker // chunk_rows
    bufs, rsems, descs = [buf0, buf1], [rsem0, rsem1], [None, None]
    for sl in range(min(2, nchunks)):
        descs[sl] = pltpu.async_copy(
            pe_hbm.at[pl.ds(base + sl * chunk_rows, chunk_rows)],
            bufs[sl], rsems[sl])
    for c in range(nchunks):
        sl = c % 2
        descs[sl].wait()
        r = base + c * chunk_rows
        for b in range(B):
            pltpu.sync_copy(bufs[sl], out_hbm.at[pl.ds(b * T + r, chunk_rows)])
        nxt = c + 2
        if nxt < nchunks:
            descs[sl] = pltpu.async_copy(
                pe_hbm.at[pl.ds(base + nxt * chunk_rows, chunk_rows)],
                bufs[sl], rsems[sl])


@functools.partial(jax.jit, static_argnums=(0, 1, 2))
def _pe_broadcast(B, T, D, pe):
    rows_per_worker = T // _NUM_WORKERS
    chunk_rows = min(rows_per_worker, 32)
    mesh = plsc.VectorSubcoreMesh(
        core_axis_name="c", subcore_axis_name="s",
        num_cores=_NUM_CORES, num_subcores=_NUM_SUBCORES)
    body = functools.partial(_pe_broadcast_body, B, T, D, rows_per_worker,
                             chunk_rows)
    out_flat = pl.kernel(
        body,
        out_type=jax.ShapeDtypeStruct((B * T, D), pe.dtype),
        mesh=mesh,
        scratch_types=[
            pltpu.VMEM((chunk_rows, D), pe.dtype),
            pltpu.VMEM((chunk_rows, D), pe.dtype),
            pltpu.SemaphoreType.DMA,
            pltpu.SemaphoreType.DMA,
        ],
    )(pe)
    return out_flat.reshape(B, T, D)


def kernel(x, pe):
    B, T, D = x.shape
    return _pe_broadcast(B, T, D, pe)


# speedup vs baseline: 1.0106x; 1.0106x over previous
import functools

import jax
import jax.numpy as jnp
from jax import lax
from jax.experimental import pallas as pl
from jax.experimental.pallas import tpu as pltpu
from jax.experimental.pallas import tpu_sc as plsc

_NUM_CORES = 2
_NUM_SUBCORES = 16
_NUM_WORKERS = _NUM_CORES * _NUM_SUBCORES


def _pe_broadcast_body(B, T, D, rows_per_worker, chunk_rows, pe_hbm, out_hbm,
                       buf_v):
    wid = lax.axis_index("s") * _NUM_CORES + lax.axis_index("c")
    base = wid * rows_per_worker
    for c in range(rows_per_worker // chunk_rows):
        r = base + c * chunk_rows
        pltpu.sync_copy(pe_hbm.at[pl.ds(r, chunk_rows)], buf_v)
        for b in range(B):
            pltpu.sync_copy(buf_v, out_hbm.at[pl.ds(b * T + r, chunk_rows)])


@functools.partial(jax.jit, static_argnums=(0, 1, 2))
def _pe_broadcast(B, T, D, pe):
    rows_per_worker = T // _NUM_WORKERS
    chunk_rows = min(rows_per_worker, 64)
    mesh = plsc.VectorSubcoreMesh(
        core_axis_name="c", subcore_axis_name="s",
        num_cores=_NUM_CORES, num_subcores=_NUM_SUBCORES)
    body = functools.partial(_pe_broadcast_body, B, T, D, rows_per_worker,
                             chunk_rows)
    out_flat = pl.kernel(
        body,
        out_type=jax.ShapeDtypeStruct((B * T, D), pe.dtype),
        mesh=mesh,
        scratch_types=[pltpu.VMEM((chunk_rows, D), pe.dtype)],
    )(pe)
    return out_flat.reshape(B, T, D)


def kernel(x, pe):
    B, T, D = x.shape
    return _pe_broadcast(B, T, D, pe)
